# Initial kernel scaffold; baseline (speedup 1.0000x reference)
import jax
import jax.numpy as jnp
from jax.experimental import pallas as pl


def kernel(x, edge_index, W, b, gamma, beta):
    raise NotImplementedError("write your pallas kernel here")


# speedup vs baseline: 6.8279x; 6.8279x over previous
import functools

import jax
import jax.numpy as jnp
from jax import lax
from jax.experimental import pallas as pl
from jax.experimental.pallas import tpu as pltpu
from jax.experimental.pallas import tpu_sc as plsc

NC = 2
NS = 16
NW = NC * NS
LANES = 16
CHUNK = 128


def _slice_copy(sub, row0, ra, last, src_of, dst_of):
    @pl.when(sub < NS - 1)
    def _():
        pltpu.sync_copy(src_of(row0, ra), dst_of(row0, ra))

    @pl.when(sub == NS - 1)
    def _():
        pltpu.sync_copy(src_of(row0, last), dst_of(row0, last))


def _deg_body(nch, ra, last, dst_hbm, zeros_hbm, ones_hbm, out_hbm, dst_v, ones_v, acc, dsem):
    c = lax.axis_index("c")
    s = lax.axis_index("s")
    wid = c * NS + s
    row0 = s * ra
    pltpu.sync_copy(dst_hbm.at[wid], dst_v)
    pltpu.sync_copy(ones_hbm, ones_v)
    _slice_copy(s, row0, ra, last,
                lambda r, n: zeros_hbm.at[pl.ds(0, n)],
                lambda r, n: acc.at[pl.ds(r, n)])
    plsc.subcore_barrier()

    def fire(j, carry):
        pltpu.async_copy(ones_v, acc.at[dst_v.at[j]], dsem, add=True)
        return carry

    lax.fori_loop(0, nch, fire, 0)

    def drain(j, carry):
        pltpu.make_async_copy(ones_v, acc.at[dst_v.at[j]], dsem).wait()
        return carry

    lax.fori_loop(0, nch, drain, 0)
    plsc.subcore_barrier()
    _slice_copy(s, row0, ra, last,
                lambda r, n: acc.at[pl.ds(r, n)],
                lambda r, n: out_hbm.at[c].at[pl.ds(r, n)])


def _msg_body(nch, ra, last, idx_hbm, hs_hbm, out_hbm,
              i0, i1, i2, i3, i4, i5, b0, b1, b2, acc,
              gs0, gs1, gs2, ss0, ss1, ss2, is0, is1, is2, is3, is4, is5):
    c = lax.axis_index("c")
    s = lax.axis_index("s")
    wid = c * NS + s
    row0 = s * ra
    base = wid * nch
    ibufs = (i0, i1, i2, i3, i4, i5)
    isems = (is0, is1, is2, is3, is4, is5)
    bufs = (b0, b1, b2)
    gsems = (gs0, gs1, gs2)
    ssems = (ss0, ss1, ss2)

    pltpu.async_copy(idx_hbm.at[base + 0], i0, is0)
    pltpu.async_copy(idx_hbm.at[base + 1], i1, is1)
    _slice_copy(s, row0, ra, last,
                lambda r, n: hs_hbm.at[pl.ds(r, n)],
                lambda r, n: acc.at[pl.ds(r, n)])
    plsc.subcore_barrier()

    pltpu.make_async_copy(idx_hbm.at[base + 0], i0, is0).wait()
    pltpu.async_copy(hs_hbm.at[i0.at[0]], b0, gs0)

    def step(t, carry):
        j = 6 * t
        for m in range(6):
            k = j + m

            @pl.when(k >= 2)
            def _():
                pltpu.make_async_copy(
                    bufs[(m - 2) % 3], acc.at[ibufs[(m - 2) % 6].at[1]],
                    ssems[(m - 2) % 3]).wait()

            @pl.when(k + 2 < nch)
            def _():
                pltpu.async_copy(idx_hbm.at[base + k + 2],
                                 ibufs[(m + 2) % 6], isems[(m + 2) % 6])

            @pl.when(k + 1 < nch)
            def _():
                pltpu.make_async_copy(idx_hbm.at[base + k + 1],
                                      ibufs[(m + 1) % 6], isems[(m + 1) % 6]).wait()
                pltpu.async_copy(hs_hbm.at[ibufs[(m + 1) % 6].at[0]],
                                 bufs[(m + 1) % 3], gsems[(m + 1) % 3])

            pltpu.make_async_copy(hs_hbm.at[ibufs[m % 6].at[0]],
                                  bufs[m % 3], gsems[m % 3]).wait()
            pltpu.async_copy(bufs[m % 3], acc.at[ibufs[m % 6].at[1]],
                             ssems[m % 3], add=True)
        return carry

    lax.fori_loop(0, nch // 6, step, 0)
    for k in (nch - 2, nch - 1):
        pltpu.make_async_copy(bufs[k % 3], acc.at[ibufs[k % 6].at[1]],
                              ssems[k % 3]).wait()
    plsc.subcore_barrier()
    _slice_copy(s, row0, ra, last,
                lambda r, n: acc.at[pl.ds(r, n)],
                lambda r, n: out_hbm.at[c].at[pl.ds(r, n)])


def _mm_body(x_ref, w_ref, degp_ref, hs_ref):
    deg = jnp.sum(degp_ref[...], axis=0)[:, 0:1] + 1.0
    dinv = lax.rsqrt(deg)
    h = jnp.dot(x_ref[...], w_ref[...], preferred_element_type=jnp.float32)
    hs_ref[...] = h * dinv


def _pre_body(aggp_ref, hs_ref, degp_ref, b_ref, pre_ref, stats_ref):
    i = pl.program_id(0)
    deg = jnp.sum(degp_ref[...], axis=0)[:, 0:1] + 1.0
    dinv = lax.rsqrt(deg)
    agg = jnp.sum(aggp_ref[...], axis=0)
    pre = (agg - hs_ref[...]) * dinv + b_ref[...]
    pre_ref[...] = pre
    ssum = jnp.sum(pre, axis=0, keepdims=True)
    ssq = jnp.sum(pre * pre, axis=0, keepdims=True)
    st = jnp.concatenate([ssum, ssq], axis=0)

    @pl.when(i == 0)
    def _():
        stats_ref[...] = st

    @pl.when(i != 0)
    def _():
        stats_ref[...] = stats_ref[...] + st


def _bn_body(n, pre_ref, stats_ref, g_ref, be_ref, out_ref):
    mean = stats_ref[0:1, :] / n
    var = stats_ref[1:2, :] / n - mean * mean
    rstd = lax.rsqrt(var + 1e-5)
    out_ref[...] = jnp.tanh((pre_ref[...] - mean) * rstd * g_ref[...] + be_ref[...])


def _row_block(n):
    for cand in (1024, 1000, 512, 500, 400, 256, 200, 128, 80, 40, 16, 8):
        if n % cand == 0 and cand % 8 == 0:
            return cand
    return 8


def kernel(x, edge_index, W, b, gamma, beta):
    N, D = x.shape
    E = edge_index.shape[1]
    idt = edge_index.dtype

    N3 = ((N + 16 + 15) // 16) * 16
    ra = ((-(-N3 // NS) + 7) // 8) * 8
    last = N3 - (NS - 1) * ra
    nch = -(-E // (NW * CHUNK))
    nch = ((nch + 5) // 6) * 6
    e_pad = NW * nch * CHUNK
    epw = nch * CHUNK
    ew = E // NW
    rem = E - ew * NW
    pad_n = epw - ew
    sinks = N + 1 + (jnp.arange(NW * pad_n, dtype=idt).reshape(NW, pad_n) % (N3 - N - 1))
    srcw = jnp.concatenate(
        [edge_index[0][:ew * NW].reshape(NW, ew),
         jnp.full((NW, pad_n), N, idt)], axis=1)
    dstw = jnp.concatenate(
        [edge_index[1][:ew * NW].reshape(NW, ew), sinks], axis=1)
    if rem:
        srcw = srcw.at[0, ew:ew + rem].set(edge_index[0][ew * NW:])
        dstw = dstw.at[0, ew:ew + rem].set(edge_index[1][ew * NW:])
    dst_p = dstw.reshape(NW, nch, CHUNK)
    idx_p = jnp.stack([srcw.reshape(NW, nch, CHUNK),
                       dstw.reshape(NW, nch, CHUNK)], axis=2
                      ).reshape(NW * nch, 2, CHUNK)
    x_p = jnp.concatenate([x, jnp.zeros((N3 - N, D), x.dtype)], axis=0)

    deg_zero = jnp.zeros((ra, D), jnp.float32)
    deg_ones = jnp.ones((CHUNK, D), jnp.float32)

    mesh = plsc.VectorSubcoreMesh(core_axis_name="c", subcore_axis_name="s")

    deg_call = pl.kernel(
        functools.partial(_deg_body, nch, ra, last),
        out_type=jax.ShapeDtypeStruct((NC, N3, D), jnp.float32),
        mesh=mesh,
        scratch_types=[
            pltpu.VMEM((nch, CHUNK), jnp.int32),
            pltpu.VMEM((CHUNK, D), jnp.float32),
            pltpu.VMEM_SHARED((N3, D), jnp.float32),
            pltpu.SemaphoreType.DMA,
        ],
    )
    degp = deg_call(dst_p, deg_zero, deg_ones)

    BR = N3 // 4 if (N3 // 4) % 8 == 0 else 16
    hs = pl.pallas_call(
        _mm_body,
        grid=(N3 // BR,),
        in_specs=[
            pl.BlockSpec((BR, D), lambda i: (i, 0)),
            pl.BlockSpec((D, D), lambda i: (0, 0)),
            pl.BlockSpec((NC, BR, D), lambda i: (0, i, 0)),
        ],
        out_specs=pl.BlockSpec((BR, D), lambda i: (i, 0)),
        out_shape=jax.ShapeDtypeStruct((N3, D), jnp.float32),
    )(x_p, W, degp)

    msg_call = pl.kernel(
        functools.partial(_msg_body, nch, ra, last),
        out_type=jax.ShapeDtypeStruct((NC, N3, D), jnp.float32),
        mesh=mesh,
        scratch_types=(
            [pltpu.VMEM((2, CHUNK), jnp.int32)] * 6
            + [pltpu.VMEM((CHUNK, D), jnp.float32)] * 3
            + [pltpu.VMEM_SHARED((N3, D), jnp.float32)]
            + [pltpu.SemaphoreType.DMA] * 12
        ),
    )
    aggp = msg_call(idx_p, hs)

    BB = _row_block(N)
    nb = N // BB
    pre, stats = pl.pallas_call(
        _pre_body,
        grid=(nb,),
        in_specs=[
            pl.BlockSpec((NC, BB, D), lambda i: (0, i, 0)),
            pl.BlockSpec((BB, D), lambda i: (i, 0)),
            pl.BlockSpec((NC, BB, D), lambda i: (0, i, 0)),
            pl.BlockSpec((1, D), lambda i: (0, 0)),
        ],
        out_specs=[
            pl.BlockSpec((BB, D), lambda i: (i, 0)),
            pl.BlockSpec((2, D), lambda i: (0, 0)),
        ],
        out_shape=[
            jax.ShapeDtypeStruct((N, D), jnp.float32),
            jax.ShapeDtypeStruct((2, D), jnp.float32),
        ],
    )(aggp, hs, degp, b.reshape(1, D))

    out = pl.pallas_call(
        functools.partial(_bn_body, float(N)),
        grid=(nb,),
        in_specs=[
            pl.BlockSpec((BB, D), lambda i: (i, 0)),
            pl.BlockSpec((2, D), lambda i: (0, 0)),
            pl.BlockSpec((1, D), lambda i: (0, 0)),
            pl.BlockSpec((1, D), lambda i: (0, 0)),
        ],
        out_specs=pl.BlockSpec((BB, D), lambda i: (i, 0)),
        out_shape=jax.ShapeDtypeStruct((N, D), jnp.float32),
    )(pre, stats, gamma.reshape(1, D), beta.reshape(1, D))
    return out


# speedup vs baseline: 13.3997x; 1.9625x over previous
import functools

import jax
import jax.numpy as jnp
from jax import lax
from jax.experimental import pallas as pl
from jax.experimental.pallas import tpu as pltpu
from jax.experimental.pallas import tpu_sc as plsc

NC = 2
NS = 16
NW = NC * NS
LANES = 16
CHUNK = 128


def _deg_body(nch, rpw, dst_hbm, zeros_hbm, ones_hbm, out_hbm, dst_v, ones_v, acc, dsem):
    c = lax.axis_index("c")
    s = lax.axis_index("s")
    wid = c * NS + s
    row0 = s * rpw
    pltpu.sync_copy(dst_hbm.at[wid], dst_v)
    pltpu.sync_copy(ones_hbm, ones_v)
    pltpu.sync_copy(zeros_hbm, acc.at[pl.ds(row0, rpw)])
    plsc.subcore_barrier()

    def fire(j, carry):
        pltpu.async_copy(ones_v, acc.at[dst_v.at[j]], dsem, add=True)
        return carry

    lax.fori_loop(0, nch, fire, 0)

    def drain(j, carry):
        pltpu.make_async_copy(ones_v, acc.at[dst_v.at[j]], dsem).wait()
        return carry

    lax.fori_loop(0, nch, drain, 0)
    plsc.subcore_barrier()
    pltpu.sync_copy(acc.at[pl.ds(row0, rpw)], out_hbm.at[c].at[pl.ds(row0, rpw)])


def _msg_body(nch, rpw, src_hbm, dst_hbm, hs_hbm, out_hbm,
              src_v, dst_v, buf0, buf1, acc, gsem0, gsem1, ssem0, ssem1):
    c = lax.axis_index("c")
    s = lax.axis_index("s")
    wid = c * NS + s
    row0 = s * rpw
    pltpu.sync_copy(hs_hbm.at[pl.ds(row0, rpw)], acc.at[pl.ds(row0, rpw)])
    plsc.subcore_barrier()

    half = nch // 2
    bufs = (buf0, buf1)
    gsems = (gsem0, gsem1)
    ssems = (ssem0, ssem1)
    for h in range(2):
        pltpu.sync_copy(src_hbm.at[2 * wid + h], src_v)
        pltpu.sync_copy(dst_hbm.at[2 * wid + h], dst_v)
        pltpu.async_copy(hs_hbm.at[src_v.at[0]], buf0, gsem0)
        pltpu.async_copy(hs_hbm.at[src_v.at[1]], buf1, gsem1)

        def step(t, carry):
            j = 2 * t
            for jj in range(2):
                k = j + jj
                pltpu.make_async_copy(hs_hbm.at[src_v.at[k]], bufs[jj], gsems[jj]).wait()
                pltpu.async_copy(bufs[jj], acc.at[dst_v.at[k]], ssems[jj], add=True)
            for jj in range(2):
                k = j + jj
                pltpu.make_async_copy(bufs[jj], acc.at[dst_v.at[k]], ssems[jj]).wait()

                @pl.when(k + 2 < half)
                def _():
                    pltpu.async_copy(hs_hbm.at[src_v.at[k + 2]], bufs[jj], gsems[jj])
            return carry

        lax.fori_loop(0, half // 2, step, 0)
    plsc.subcore_barrier()
    pltpu.sync_copy(acc.at[pl.ds(row0, rpw)], out_hbm.at[c].at[pl.ds(row0, rpw)])


def _mm_body(x_ref, w_ref, degp_ref, hs_ref):
    deg = jnp.sum(degp_ref[...], axis=0)[:, 0:1] + 1.0
    dinv = lax.rsqrt(deg)
    h = jnp.dot(x_ref[...], w_ref[...], preferred_element_type=jnp.float32)
    hs_ref[...] = h * dinv


def _pre_body(aggp_ref, hs_ref, degp_ref, b_ref, pre_ref, stats_ref):
    i = pl.program_id(0)
    deg = jnp.sum(degp_ref[...], axis=0)[:, 0:1] + 1.0
    dinv = lax.rsqrt(deg)
    agg = jnp.sum(aggp_ref[...], axis=0)
    pre = (agg - hs_ref[...]) * dinv + b_ref[...]
    pre_ref[...] = pre
    ssum = jnp.sum(pre, axis=0, keepdims=True)
    ssq = jnp.sum(pre * pre, axis=0, keepdims=True)
    st = jnp.concatenate([ssum, ssq], axis=0)

    @pl.when(i == 0)
    def _():
        stats_ref[...] = st

    @pl.when(i != 0)
    def _():
        stats_ref[...] = stats_ref[...] + st


def _bn_body(n, pre_ref, stats_ref, g_ref, be_ref, out_ref):
    mean = stats_ref[0:1, :] / n
    var = stats_ref[1:2, :] / n - mean * mean
    rstd = lax.rsqrt(var + 1e-5)
    out_ref[...] = jnp.tanh((pre_ref[...] - mean) * rstd * g_ref[...] + be_ref[...])


def _row_block(n):
    for cand in (1024, 1000, 512, 500, 400, 256, 200, 128, 80, 40, 16, 8):
        if n % cand == 0 and cand % 8 == 0:
            return cand
    return 8


def kernel(x, edge_index, W, b, gamma, beta):
    N, D = x.shape
    E = edge_index.shape[1]
    idt = edge_index.dtype

    N3 = ((N + 1 + 255) // 256) * 256
    rpw = N3 // NS
    nch = -(-E // (NW * CHUNK))
    nch = ((nch + 3) // 4) * 4
    e_pad = NW * nch * CHUNK
    epw = nch * CHUNK
    ew = E // NW
    rem = E - ew * NW
    pad_n = epw - ew
    sinks = N + 1 + (jnp.arange(NW * pad_n, dtype=idt).reshape(NW, pad_n) % (N3 - N - 1))
    srcw = jnp.concatenate(
        [edge_index[0][:ew * NW].reshape(NW, ew),
         jnp.full((NW, pad_n), N, idt)], axis=1)
    dstw = jnp.concatenate(
        [edge_index[1][:ew * NW].reshape(NW, ew), sinks], axis=1)
    if rem:
        srcw = srcw.at[0, ew:ew + rem].set(edge_index[0][ew * NW:])
        dstw = dstw.at[0, ew:ew + rem].set(edge_index[1][ew * NW:])
    src_p = srcw.reshape(NW, nch, CHUNK)
    dst_p = dstw.reshape(NW, nch, CHUNK)
    src_p2 = srcw.reshape(NW * 2, nch // 2, CHUNK)
    dst_p2 = dstw.reshape(NW * 2, nch // 2, CHUNK)
    x_p = jnp.concatenate([x, jnp.zeros((N3 - N, D), x.dtype)], axis=0)

    deg_zero = jnp.zeros((rpw, D), jnp.float32)
    deg_ones = jnp.ones((CHUNK, D), jnp.float32)

    mesh = plsc.VectorSubcoreMesh(core_axis_name="c", subcore_axis_name="s")

    deg_call = pl.kernel(
        functools.partial(_deg_body, nch, rpw),
        out_type=jax.ShapeDtypeStruct((NC, N3, D), jnp.float32),
        mesh=mesh,
        scratch_types=[
            pltpu.VMEM((nch, CHUNK), jnp.int32),
            pltpu.VMEM((CHUNK, D), jnp.float32),
            pltpu.VMEM_SHARED((N3, D), jnp.float32),
            pltpu.SemaphoreType.DMA,
        ],
    )
    degp = deg_call(dst_p, deg_zero, deg_ones)

    BR = 256
    hs = pl.pallas_call(
        _mm_body,
        grid=(N3 // BR,),
        in_specs=[
            pl.BlockSpec((BR, D), lambda i: (i, 0)),
            pl.BlockSpec((D, D), lambda i: (0, 0)),
            pl.BlockSpec((NC, BR, D), lambda i: (0, i, 0)),
        ],
        out_specs=pl.BlockSpec((BR, D), lambda i: (i, 0)),
        out_shape=jax.ShapeDtypeStruct((N3, D), jnp.float32),
    )(x_p, W, degp)

    msg_call = pl.kernel(
        functools.partial(_msg_body, nch, rpw),
        out_type=jax.ShapeDtypeStruct((NC, N3, D), jnp.float32),
        mesh=mesh,
        scratch_types=[
            pltpu.VMEM((nch // 2, CHUNK), jnp.int32),
            pltpu.VMEM((nch // 2, CHUNK), jnp.int32),
            pltpu.VMEM((CHUNK, D), jnp.float32),
            pltpu.VMEM((CHUNK, D), jnp.float32),
            pltpu.VMEM_SHARED((N3, D), jnp.float32),
            pltpu.SemaphoreType.DMA,
            pltpu.SemaphoreType.DMA,
            pltpu.SemaphoreType.DMA,
            pltpu.SemaphoreType.DMA,
        ],
    )
    aggp = msg_call(src_p2, dst_p2, hs)

    BB = _row_block(N)
    nb = N // BB
    pre, stats = pl.pallas_call(
        _pre_body,
        grid=(nb,),
        in_specs=[
            pl.BlockSpec((NC, BB, D), lambda i: (0, i, 0)),
            pl.BlockSpec((BB, D), lambda i: (i, 0)),
            pl.BlockSpec((NC, BB, D), lambda i: (0, i, 0)),
            pl.BlockSpec((1, D), lambda i: (0, 0)),
        ],
        out_specs=[
            pl.BlockSpec((BB, D), lambda i: (i, 0)),
            pl.BlockSpec((2, D), lambda i: (0, 0)),
        ],
        out_shape=[
            jax.ShapeDtypeStruct((N, D), jnp.float32),
            jax.ShapeDtypeStruct((2, D), jnp.float32),
        ],
    )(aggp, hs, degp, b.reshape(1, D))

    out = pl.pallas_call(
        functools.partial(_bn_body, float(N)),
        grid=(nb,),
        in_specs=[
            pl.BlockSpec((BB, D), lambda i: (i, 0)),
            pl.BlockSpec((2, D), lambda i: (0, 0)),
            pl.BlockSpec((1, D), lambda i: (0, 0)),
            pl.BlockSpec((1, D), lambda i: (0, 0)),
        ],
        out_specs=pl.BlockSpec((BB, D), lambda i: (i, 0)),
        out_shape=jax.ShapeDtypeStruct((N, D), jnp.float32),
    )(pre, stats, gamma.reshape(1, D), beta.reshape(1, D))
    return out
